# Initial kernel scaffold; baseline (speedup 1.0000x reference)
import jax
import jax.numpy as jnp
from jax.experimental import pallas as pl


def kernel(x, emb):
    raise NotImplementedError("write your pallas kernel here")


# speedup vs baseline: 1.7697x; 1.7697x over previous
import jax
import jax.numpy as jnp
from jax.experimental import pallas as pl

_SEG = 2048
_R = 512


def _add_kernel(x_ref, e_ref, o_ref):
    seg = (pl.program_id(0) * _R) // _SEG
    o_ref[...] = x_ref[...] + e_ref[seg, :][None, None, :]


def kernel(x, emb):
    n, b, d = x.shape
    grid = (n // _R,)
    return pl.pallas_call(
        _add_kernel,
        grid=grid,
        in_specs=[
            pl.BlockSpec((_R, b, d), lambda i: (i, 0, 0)),
            pl.BlockSpec(emb.shape, lambda i: (0, 0)),
        ],
        out_specs=pl.BlockSpec((_R, b, d), lambda i: (i, 0, 0)),
        out_shape=jax.ShapeDtypeStruct(x.shape, x.dtype),
    )(x, emb)
